# Initial kernel scaffold; baseline (speedup 1.0000x reference)
import jax
import jax.numpy as jnp
from jax.experimental import pallas as pl


def kernel(tokens, scores):
    raise NotImplementedError("write your pallas kernel here")


# speedup vs baseline: 1.1424x; 1.1424x over previous
import functools

import jax
import jax.numpy as jnp
from jax import lax
from jax.experimental import pallas as pl
from jax.experimental.pallas import tpu as pltpu
from jax.experimental.pallas import tpu_sc as plsc

N_HEADS = 16
N_TOK = 2048
N_SEL = N_TOK // 2
ROW_TILE = 512
N_ROW_TILES = N_TOK // ROW_TILE
N_BATCH = 4
D_MODEL = 768
J_CHUNK = 512


def _reduce_select_body(s_ref, ids_ref, row_acc, col_acc):
    i = pl.program_id(0)
    h = pl.program_id(1)
    x = s_ref[0]
    rp = jnp.sum(x, axis=1)
    cp = jnp.sum(x, axis=0)

    @pl.when(h == 0)
    def _():
        row_acc[0, pl.ds(i * ROW_TILE, ROW_TILE)] = rp

    @pl.when(h != 0)
    def _():
        row_acc[0, pl.ds(i * ROW_TILE, ROW_TILE)] = (
            row_acc[0, pl.ds(i * ROW_TILE, ROW_TILE)] + rp
        )

    first = jnp.logical_and(i == 0, h == 0)

    @pl.when(first)
    def _():
        col_acc[0, :] = cp

    @pl.when(jnp.logical_not(first))
    def _():
        col_acc[0, :] = col_acc[0, :] + cp

    @pl.when(jnp.logical_and(i == N_ROW_TILES - 1, h == N_HEADS - 1))
    def _():
        inv = 1.0 / (N_HEADS * N_TOK)
        s1 = row_acc[0, :] * inv
        s2 = col_acc[0, :] * inv
        m1 = jnp.sum(s1) * (1.0 / N_TOK)
        m2 = jnp.sum(s2) * (1.0 / N_TOK)
        v1 = jnp.sum((s1 - m1) ** 2)
        v2 = jnp.sum((s2 - m2) ** 2)
        sig_row = jnp.where(v1 > v2, s1, s2).reshape(1, N_TOK)
        sig_col = sig_row.reshape(N_TOK, 1)

        ii = lax.broadcasted_iota(jnp.int32, (J_CHUNK, N_TOK), 1)
        jj0 = lax.broadcasted_iota(jnp.int32, (J_CHUNK, N_TOK), 0)
        rank = jnp.zeros((1, N_TOK), jnp.float32)
        for j0 in range(0, N_TOK, J_CHUNK):
            sj = lax.slice(sig_col, (j0, 0), (j0 + J_CHUNK, 1))
            beats = (sj > sig_row) | ((sj == sig_row) & (jj0 + j0 < ii))
            rank = rank + jnp.sum(beats.astype(jnp.float32), axis=0,
                                  keepdims=True)
        sel_row = (rank < float(N_SEL)).astype(jnp.float32)
        sel_col = sel_row.reshape(N_TOK, 1)

        pos = jnp.zeros((1, N_TOK), jnp.float32)
        for j0 in range(0, N_TOK, J_CHUNK):
            sj = lax.slice(sel_col, (j0, 0), (j0 + J_CHUNK, 1))
            before = sj * ((jj0 + j0) < ii).astype(jnp.float32)
            pos = pos + jnp.sum(before, axis=0, keepdims=True)
        pos_col = pos.reshape(N_TOK, 1)

        pp = lax.broadcasted_iota(jnp.int32, (J_CHUNK, N_SEL), 1
                                  ).astype(jnp.float32)
        ids = jnp.zeros((1, N_SEL), jnp.float32)
        for i0 in range(0, N_TOK, J_CHUNK):
            si = lax.slice(sel_col, (i0, 0), (i0 + J_CHUNK, 1))
            pi = lax.slice(pos_col, (i0, 0), (i0 + J_CHUNK, 1))
            ival = lax.broadcasted_iota(jnp.int32, (J_CHUNK, N_SEL), 0
                                        ).astype(jnp.float32) + i0
            contrib = si * (pi == pp).astype(jnp.float32) * ival
            ids = ids + jnp.sum(contrib, axis=0, keepdims=True)
        ids = ids.astype(jnp.int32)
        for b in range(N_BATCH):
            ids_ref[0, pl.ds(b * N_SEL, N_SEL)] = ids[0] + b * N_TOK


def _select_ids(scores):
    return pl.pallas_call(
        _reduce_select_body,
        grid=(N_ROW_TILES, N_HEADS),
        in_specs=[
            pl.BlockSpec((1, ROW_TILE, N_TOK), lambda i, h: (h, i, 0)),
        ],
        out_specs=pl.BlockSpec((1, N_BATCH * N_SEL), lambda i, h: (0, 0)),
        out_shape=jax.ShapeDtypeStruct((1, N_BATCH * N_SEL), jnp.int32),
        scratch_shapes=[
            pltpu.VMEM((1, N_TOK), jnp.float32),
            pltpu.VMEM((1, N_TOK), jnp.float32),
        ],
    )(scores)


_NC, _NS = 2, 16
_B_PER_W = (N_BATCH * N_SEL) // (_NC * _NS)


@functools.cache
def _sc_gather_fn():
    mesh = plsc.VectorSubcoreMesh(core_axis_name="c", subcore_axis_name="s")

    @functools.partial(
        pl.kernel,
        mesh=mesh,
        out_type=jax.ShapeDtypeStruct((N_BATCH * N_SEL, D_MODEL), jnp.float32),
        scratch_types=[
            pltpu.VMEM((_B_PER_W,), jnp.int32),
            pltpu.VMEM((_B_PER_W, D_MODEL), jnp.float32),
            pltpu.SemaphoreType.DMA,
        ],
    )
    def _sc_gather(table_hbm, idx_hbm, out_hbm, idx_v, rows_v, sem):
        wid = lax.axis_index("s") * _NC + lax.axis_index("c")
        base = wid * _B_PER_W
        pltpu.sync_copy(idx_hbm.at[pl.ds(base, _B_PER_W)], idx_v)
        pltpu.async_copy(table_hbm.at[idx_v], rows_v, sem).wait()
        pltpu.sync_copy(rows_v, out_hbm.at[pl.ds(base, _B_PER_W)])

    return _sc_gather


def kernel(tokens, scores):
    ids4 = _select_ids(scores).reshape(N_BATCH * N_SEL)
    table = tokens.reshape(N_BATCH * N_TOK, D_MODEL)
    out = _sc_gather_fn()(table, ids4)
    return out.reshape(N_BATCH, N_SEL, D_MODEL)
